# Initial kernel scaffold; baseline (speedup 1.0000x reference)
import functools

import jax
import jax.numpy as jnp
from jax import lax
from jax.experimental import pallas as pl
from jax.experimental.pallas import tpu as pltpu
from jax.experimental.pallas import tpu_sc as plsc

N = 10000
D = 128
E = 320000

NP = 10240
EP = 327680
EROWS = EP // 128
RPT = EROWS // 32
CR = 4
NCHUNK = RPT // CR
TSLICE = NP // 16

_MESH = plsc.VectorSubcoreMesh(
    core_axis_name="c", subcore_axis_name="s", num_cores=2, num_subcores=16)


def _edge_body(h_hbm, as_hbm, ad_hbm, src_hbm, dst_hbm, z128_hbm, z1_hbm,
               acc_out, den_out,
               as_v, ad_v, src_v, dst_v, ex_v, rows_v, acc_sh, den_sh, sem):
    c = lax.axis_index("c")
    s = lax.axis_index("s")
    w = c * 16 + s

    pltpu.sync_copy(as_hbm, as_v)
    pltpu.sync_copy(ad_hbm, ad_v)
    t0 = s * TSLICE
    pltpu.sync_copy(z128_hbm.at[pl.ds(t0, TSLICE)], acc_sh.at[pl.ds(t0, TSLICE)])
    pltpu.sync_copy(z1_hbm.at[pl.ds(t0, TSLICE)], den_sh.at[pl.ds(t0, TSLICE)])
    plsc.subcore_barrier()

    row0 = w * RPT

    def chunk(k, carry):
        base = row0 + k * CR
        pltpu.sync_copy(src_hbm.at[pl.ds(base, CR)], src_v)
        pltpu.sync_copy(dst_hbm.at[pl.ds(base, CR)], dst_v)
        descs = [
            pltpu.async_copy(h_hbm.at[src_v.at[r]],
                             rows_v.at[pl.ds(r * 128, 128)], sem)
            for r in range(CR)
        ]
        for d in descs:
            d.wait()
        for r in range(CR):
            for i in range(8):
                sl = pl.ds(i * 16, 16)
                s16 = src_v[r, sl]
                d16 = dst_v[r, sl]
                e = plsc.load_gather(as_v, [s16]) + plsc.load_gather(ad_v, [d16])
                e = jnp.where(e >= 0.0, e, 0.2 * e)
                ex_v[pl.ds(r * 128 + i * 16, 16)] = jnp.exp(e)

        def scale(j, carry2):
            exj = ex_v[j]
            for kk in range(8):
                sl = pl.ds(kk * 16, 16)
                rows_v[j, sl] = rows_v[j, sl] * exj
            return carry2

        lax.fori_loop(0, CR * 128, scale, 0)
        for r in range(CR):
            pltpu.sync_copy(rows_v.at[pl.ds(r * 128, 128)],
                            acc_sh.at[dst_v.at[r]], add=True)
            pltpu.sync_copy(ex_v.at[pl.ds(r * 128, 128)],
                            den_sh.at[dst_v.at[r]], add=True)
        return carry

    lax.fori_loop(0, NCHUNK, chunk, 0)
    plsc.subcore_barrier()
    pltpu.sync_copy(acc_sh.at[pl.ds(t0, TSLICE)],
                    acc_out.at[pl.ds(c * NP + t0, TSLICE)])
    pltpu.sync_copy(den_sh.at[pl.ds(t0, TSLICE)],
                    den_out.at[pl.ds(c * NP + t0, TSLICE)])


_edge_pass = pl.kernel(
    _edge_body,
    out_type=(
        jax.ShapeDtypeStruct((2 * NP, D), jnp.float32),
        jax.ShapeDtypeStruct((2 * NP,), jnp.float32),
    ),
    mesh=_MESH,
    scratch_types=(
        pltpu.VMEM((NP,), jnp.float32),
        pltpu.VMEM((NP,), jnp.float32),
        pltpu.VMEM((CR, 128), jnp.int32),
        pltpu.VMEM((CR, 128), jnp.int32),
        pltpu.VMEM((CR * 128,), jnp.float32),
        pltpu.VMEM((CR * 128, D), jnp.float32),
        pltpu.VMEM_SHARED((NP, D), jnp.float32),
        pltpu.VMEM_SHARED((NP,), jnp.float32),
        pltpu.SemaphoreType.DMA,
    ),
)


def _proj_body(x_ref, w_ref, avs_ref, avd_ref, h_ref, as_ref, ad_ref):
    h = jnp.dot(x_ref[...], w_ref[...], preferred_element_type=jnp.float32)
    h_ref[...] = h
    as_ref[...] = jnp.dot(h, avs_ref[...], preferred_element_type=jnp.float32)
    ad_ref[...] = jnp.dot(h, avd_ref[...], preferred_element_type=jnp.float32)


_proj = pl.pallas_call(
    _proj_body,
    out_shape=(
        jax.ShapeDtypeStruct((NP, D), jnp.float32),
        jax.ShapeDtypeStruct((NP, 1), jnp.float32),
        jax.ShapeDtypeStruct((NP, 1), jnp.float32),
    ),
)


def _combine(h, acc, den, as_c, ad_c):
    e = as_c + ad_c
    e = jnp.where(e >= 0.0, e, 0.2 * e)
    exs = jnp.exp(e)
    num = acc[0:NP] + acc[NP:2 * NP] + exs * h
    dsum = den[0:NP] + den[NP:2 * NP] + exs + 1e-16
    return num / dsum


def _comb_proj_body(h_ref, acc_ref, den_ref, as_ref, ad_ref, b_ref,
                    w_ref, avs_ref, avd_ref, h2_ref, as2_ref, ad2_ref):
    o = _combine(h_ref[...], acc_ref[...], den_ref[...],
                 as_ref[...], ad_ref[...]) + b_ref[...]
    o = jnp.where(o > 0.0, o, jnp.exp(o) - 1.0)
    h2 = jnp.dot(o, w_ref[...], preferred_element_type=jnp.float32)
    h2_ref[...] = h2
    as2_ref[...] = jnp.dot(h2, avs_ref[...], preferred_element_type=jnp.float32)
    ad2_ref[...] = jnp.dot(h2, avd_ref[...], preferred_element_type=jnp.float32)


_comb_proj = pl.pallas_call(
    _comb_proj_body,
    out_shape=(
        jax.ShapeDtypeStruct((NP, D), jnp.float32),
        jax.ShapeDtypeStruct((NP, 1), jnp.float32),
        jax.ShapeDtypeStruct((NP, 1), jnp.float32),
    ),
)


def _final_body(h_ref, acc_ref, den_ref, as_ref, ad_ref, b_ref, out_ref):
    out_ref[...] = _combine(h_ref[...], acc_ref[...], den_ref[...],
                            as_ref[...], ad_ref[...]) + b_ref[...]


_final = pl.pallas_call(
    _final_body,
    out_shape=jax.ShapeDtypeStruct((NP, D), jnp.float32),
)


def kernel(x, edge_index, W1, a_src1, a_dst1, b1, W2, a_src2, a_dst2, b2):
    xp = jnp.pad(x, ((0, NP - N), (0, 0)))
    src = edge_index[0].astype(jnp.int32)
    dst = edge_index[1].astype(jnp.int32)
    sent = jnp.full((EP - E,), NP - 1, jnp.int32)
    src2d = jnp.concatenate([src, sent]).reshape(EROWS, 128)
    dst2d = jnp.concatenate([dst, sent]).reshape(EROWS, 128)
    z128 = jnp.zeros((NP, D), jnp.float32)
    z1 = jnp.zeros((NP,), jnp.float32)

    avs1 = a_src1.reshape(D, 1)
    avd1 = a_dst1.reshape(D, 1)
    avs2 = a_src2.reshape(D, 1)
    avd2 = a_dst2.reshape(D, 1)

    h1, as1, ad1 = _proj(xp, W1, avs1, avd1)
    acc1, den1 = _edge_pass(h1, as1.reshape(NP), ad1.reshape(NP),
                            src2d, dst2d, z128, z1)
    h2, as2, ad2 = _comb_proj(h1, acc1, den1.reshape(2 * NP, 1), as1, ad1,
                              b1.reshape(1, D), W2, avs2, avd2)
    acc2, den2 = _edge_pass(h2, as2.reshape(NP), ad2.reshape(NP),
                            src2d, dst2d, z128, z1)
    out = _final(h2, acc2, den2.reshape(2 * NP, 1), as2, ad2,
                 b2.reshape(1, D))
    return out[:N]


# speedup vs baseline: 16.3714x; 16.3714x over previous
import jax
import jax.numpy as jnp
from jax import lax
from jax.experimental import pallas as pl
from jax.experimental.pallas import tpu as pltpu
from jax.experimental.pallas import tpu_sc as plsc

N = 10000
D = 128
E = 320000

NP = 10240
EP = 327680
EROWS = EP // 128
RPT = EROWS // 32
CR = 1
NCHUNK = RPT // CR
TSLICE = NP // 16

RB = 1280
GRID = NP // RB

_MESH = plsc.VectorSubcoreMesh(
    core_axis_name="c", subcore_axis_name="s", num_cores=2, num_subcores=16)


def _edge_body(h_hbm, as_hbm, ad_hbm, src_hbm, dst_hbm, z128_hbm, z1_hbm,
               acc0_out, acc1_out, den0_out, den1_out,
               as_v, ad_v, src_v, dst_v, ex_v, rows_v, acc_sh, den_sh, sem):
    c = lax.axis_index("c")
    s = lax.axis_index("s")
    w = c * 16 + s

    pltpu.sync_copy(as_hbm, as_v)
    pltpu.sync_copy(ad_hbm, ad_v)
    t0 = s * TSLICE
    pltpu.sync_copy(z128_hbm.at[pl.ds(t0, TSLICE)], acc_sh.at[pl.ds(t0, TSLICE)])
    pltpu.sync_copy(z1_hbm.at[pl.ds(t0, TSLICE)], den_sh.at[pl.ds(t0, TSLICE)])
    plsc.subcore_barrier()

    row0 = w * RPT

    def chunk(k, carry):
        base = row0 + k * CR
        pltpu.sync_copy(src_hbm.at[pl.ds(base, CR)], src_v)
        pltpu.sync_copy(dst_hbm.at[pl.ds(base, CR)], dst_v)
        descs = [
            pltpu.async_copy(h_hbm.at[src_v.at[r]],
                             rows_v.at[pl.ds(r * 128, 128)], sem)
            for r in range(CR)
        ]
        for dsc in descs:
            dsc.wait()
        for r in range(CR):
            for i in range(8):
                sl = pl.ds(i * 16, 16)
                s16 = src_v[r, sl]
                d16 = dst_v[r, sl]
                e = plsc.load_gather(as_v, [s16]) + plsc.load_gather(ad_v, [d16])
                e = jnp.where(e >= 0.0, e, 0.2 * e)
                ex_v[pl.ds(r * 128 + i * 16, 16)] = jnp.exp(e)

        def scale(jg, carry2):
            ex16 = ex_v[pl.ds(jg * 16, 16)]
            for l in range(16):
                j = jg * 16 + l
                exj = ex16[l]
                for kk in range(8):
                    sl = pl.ds(kk * 16, 16)
                    rows_v[j, sl] = rows_v[j, sl] * exj
            return carry2

        lax.fori_loop(0, CR * 8, scale, 0)
        for r in range(CR):
            pltpu.sync_copy(rows_v.at[pl.ds(r * 128, 128)],
                            acc_sh.at[dst_v.at[r]], add=True)
            pltpu.sync_copy(ex_v.at[pl.ds(r * 128, 128)],
                            den_sh.at[dst_v.at[r]], add=True)
        return carry

    lax.fori_loop(0, NCHUNK, chunk, 0)
    plsc.subcore_barrier()

    @pl.when(c == 0)
    def _():
        pltpu.sync_copy(acc_sh.at[pl.ds(t0, TSLICE)],
                        acc0_out.at[pl.ds(t0, TSLICE)])
        pltpu.sync_copy(den_sh.at[pl.ds(t0, TSLICE)],
                        den0_out.at[pl.ds(t0, TSLICE)])

    @pl.when(c == 1)
    def _():
        pltpu.sync_copy(acc_sh.at[pl.ds(t0, TSLICE)],
                        acc1_out.at[pl.ds(t0, TSLICE)])
        pltpu.sync_copy(den_sh.at[pl.ds(t0, TSLICE)],
                        den1_out.at[pl.ds(t0, TSLICE)])


_edge_pass = pl.kernel(
    _edge_body,
    out_type=(
        jax.ShapeDtypeStruct((NP, D), jnp.float32),
        jax.ShapeDtypeStruct((NP, D), jnp.float32),
        jax.ShapeDtypeStruct((NP,), jnp.float32),
        jax.ShapeDtypeStruct((NP,), jnp.float32),
    ),
    mesh=_MESH,
    compiler_params=pltpu.CompilerParams(needs_layout_passes=False),
    scratch_types=(
        pltpu.VMEM((NP,), jnp.float32),
        pltpu.VMEM((NP,), jnp.float32),
        pltpu.VMEM((CR, 128), jnp.int32),
        pltpu.VMEM((CR, 128), jnp.int32),
        pltpu.VMEM((CR * 128,), jnp.float32),
        pltpu.VMEM((CR * 128, D), jnp.float32),
        pltpu.VMEM_SHARED((NP, D), jnp.float32),
        pltpu.VMEM_SHARED((NP,), jnp.float32),
        pltpu.SemaphoreType.DMA,
    ),
)

_row_spec = pl.BlockSpec((RB, D), lambda i: (i, 0))
_col_spec = pl.BlockSpec((RB, 1), lambda i: (i, 0))


def _full_spec(r, c):
    return pl.BlockSpec((r, c), lambda i: (0, 0))


def _proj_body(x_ref, w_ref, avs_ref, avd_ref, h_ref, as_ref, ad_ref):
    h = jnp.dot(x_ref[...], w_ref[...], preferred_element_type=jnp.float32)
    h_ref[...] = h
    as_ref[...] = jnp.dot(h, avs_ref[...], preferred_element_type=jnp.float32)
    ad_ref[...] = jnp.dot(h, avd_ref[...], preferred_element_type=jnp.float32)


_proj = pl.pallas_call(
    _proj_body,
    grid=(GRID,),
    in_specs=[_row_spec, _full_spec(D, D), _full_spec(D, 1), _full_spec(D, 1)],
    out_specs=(_row_spec, _col_spec, _col_spec),
    out_shape=(
        jax.ShapeDtypeStruct((NP, D), jnp.float32),
        jax.ShapeDtypeStruct((NP, 1), jnp.float32),
        jax.ShapeDtypeStruct((NP, 1), jnp.float32),
    ),
)


def _combine(h, acc0, acc1, den0, den1, as_c, ad_c):
    e = as_c + ad_c
    e = jnp.where(e >= 0.0, e, 0.2 * e)
    exs = jnp.exp(e)
    num = acc0 + acc1 + exs * h
    dsum = den0 + den1 + exs + 1e-16
    return num / dsum


def _comb_proj_body(h_ref, acc0_ref, acc1_ref, den0_ref, den1_ref,
                    as_ref, ad_ref, b_ref,
                    w_ref, avs_ref, avd_ref, h2_ref, as2_ref, ad2_ref):
    o = _combine(h_ref[...], acc0_ref[...], acc1_ref[...], den0_ref[...],
                 den1_ref[...], as_ref[...], ad_ref[...]) + b_ref[...]
    o = jnp.where(o > 0.0, o, jnp.exp(o) - 1.0)
    h2 = jnp.dot(o, w_ref[...], preferred_element_type=jnp.float32)
    h2_ref[...] = h2
    as2_ref[...] = jnp.dot(h2, avs_ref[...], preferred_element_type=jnp.float32)
    ad2_ref[...] = jnp.dot(h2, avd_ref[...], preferred_element_type=jnp.float32)


_comb_proj = pl.pallas_call(
    _comb_proj_body,
    grid=(GRID,),
    in_specs=[_row_spec, _row_spec, _row_spec, _col_spec, _col_spec,
              _col_spec, _col_spec, _full_spec(1, D),
              _full_spec(D, D), _full_spec(D, 1), _full_spec(D, 1)],
    out_specs=(_row_spec, _col_spec, _col_spec),
    out_shape=(
        jax.ShapeDtypeStruct((NP, D), jnp.float32),
        jax.ShapeDtypeStruct((NP, 1), jnp.float32),
        jax.ShapeDtypeStruct((NP, 1), jnp.float32),
    ),
)


def _final_body(h_ref, acc0_ref, acc1_ref, den0_ref, den1_ref,
                as_ref, ad_ref, b_ref, out_ref):
    out_ref[...] = _combine(
        h_ref[...], acc0_ref[...], acc1_ref[...], den0_ref[...],
        den1_ref[...], as_ref[...], ad_ref[...]) + b_ref[...]


_final = pl.pallas_call(
    _final_body,
    grid=(GRID,),
    in_specs=[_row_spec, _row_spec, _row_spec, _col_spec, _col_spec,
              _col_spec, _col_spec, _full_spec(1, D)],
    out_specs=_row_spec,
    out_shape=jax.ShapeDtypeStruct((NP, D), jnp.float32),
)


def kernel(x, edge_index, W1, a_src1, a_dst1, b1, W2, a_src2, a_dst2, b2):
    xp = jnp.pad(x, ((0, NP - N), (0, 0)))
    src = edge_index[0].astype(jnp.int32)
    dst = edge_index[1].astype(jnp.int32)
    sent = jnp.full((EP - E,), NP - 1, jnp.int32)
    src2d = jnp.concatenate([src, sent]).reshape(EROWS, 128)
    dst2d = jnp.concatenate([dst, sent]).reshape(EROWS, 128)
    z128 = jnp.zeros((NP, D), jnp.float32)
    z1 = jnp.zeros((NP,), jnp.float32)

    avs1 = a_src1.reshape(D, 1)
    avd1 = a_dst1.reshape(D, 1)
    avs2 = a_src2.reshape(D, 1)
    avd2 = a_dst2.reshape(D, 1)

    h1, as1, ad1 = _proj(xp, W1, avs1, avd1)
    acc10, acc11, den10, den11 = _edge_pass(
        h1, as1.reshape(NP), ad1.reshape(NP), src2d, dst2d, z128, z1)
    h2, as2, ad2 = _comb_proj(
        h1, acc10, acc11, den10.reshape(NP, 1), den11.reshape(NP, 1),
        as1, ad1, b1.reshape(1, D), W2, avs2, avd2)
    acc20, acc21, den20, den21 = _edge_pass(
        h2, as2.reshape(NP), ad2.reshape(NP), src2d, dst2d, z128, z1)
    out = _final(h2, acc20, acc21, den20.reshape(NP, 1), den21.reshape(NP, 1),
                 as2, ad2, b2.reshape(1, D))
    return out[:N]


# speedup vs baseline: 21.2698x; 1.2992x over previous
import jax
import jax.numpy as jnp
from jax import lax
from jax.experimental import pallas as pl
from jax.experimental.pallas import tpu as pltpu
from jax.experimental.pallas import tpu_sc as plsc

N = 10000
D = 128
E = 320000

NP = 10240
EP = 327680
EROWS = EP // 128
RPT = EROWS // 32
CR = 1
NCHUNK = RPT // CR
TSLICE = NP // 16

RB = 1280
GRID = NP // RB

_MESH = plsc.VectorSubcoreMesh(
    core_axis_name="c", subcore_axis_name="s", num_cores=2, num_subcores=16)


def _edge_body(h_hbm, as_hbm, ad_hbm, src_hbm, dst_hbm, z128_hbm, z1_hbm,
               acc0_out, acc1_out, den0_out, den1_out,
               src_v, dst_v, asg_v, adg_v, ex_v, rows_v, acc_sh, den_sh,
               sem0, sem1):
    c = lax.axis_index("c")
    s = lax.axis_index("s")
    w = c * 16 + s

    t0 = s * TSLICE
    pltpu.sync_copy(z128_hbm.at[pl.ds(t0, TSLICE)], acc_sh.at[pl.ds(t0, TSLICE)])
    pltpu.sync_copy(z1_hbm.at[pl.ds(t0, TSLICE)], den_sh.at[pl.ds(t0, TSLICE)])
    plsc.subcore_barrier()

    row0 = w * RPT
    sems = (sem0, sem1)

    def fire(b, k):
        pltpu.sync_copy(src_hbm.at[row0 + k], src_v.at[b])
        pltpu.sync_copy(dst_hbm.at[row0 + k], dst_v.at[b])
        pltpu.async_copy(h_hbm.at[src_v.at[b]], rows_v.at[b], sems[b])
        pltpu.async_copy(as_hbm.at[src_v.at[b]], asg_v.at[b], sems[b])
        pltpu.async_copy(ad_hbm.at[dst_v.at[b]], adg_v.at[b], sems[b])

    def drain(b):
        pltpu.make_async_copy(h_hbm.at[src_v.at[b]], rows_v.at[b],
                              sems[b]).wait()
        pltpu.make_async_copy(as_hbm.at[src_v.at[b]], asg_v.at[b],
                              sems[b]).wait()
        pltpu.make_async_copy(ad_hbm.at[dst_v.at[b]], adg_v.at[b],
                              sems[b]).wait()

    def process(b):
        for i in range(8):
            sl = pl.ds(i * 16, 16)
            e = asg_v[b, sl] + adg_v[b, sl]
            e = jnp.where(e >= 0.0, e, 0.2 * e)
            ex_v[b, sl] = jnp.exp(e)

        def scale(jg, carry2):
            ex16 = ex_v[b, pl.ds(jg * 16, 16)]
            for l in range(16):
                j = jg * 16 + l
                exj = ex16[l]
                for kk in range(8):
                    sl = pl.ds(kk * 16, 16)
                    rows_v[b, j, sl] = rows_v[b, j, sl] * exj
            return carry2

        lax.fori_loop(0, 8, scale, 0, unroll=2)
        pltpu.sync_copy(rows_v.at[b], acc_sh.at[dst_v.at[b]], add=True)
        pltpu.sync_copy(ex_v.at[b], den_sh.at[dst_v.at[b]], add=True)

    fire(0, 0)
    fire(1, 1)

    def step(k2, carry):
        k = k2 * 2
        for b in range(2):
            drain(b)
            process(b)
            fire(b, k + b + 2)
        return carry

    lax.fori_loop(0, NCHUNK // 2, step, 0)
    drain(0)
    drain(1)
    plsc.subcore_barrier()

    @pl.when(c == 0)
    def _():
        pltpu.sync_copy(acc_sh.at[pl.ds(t0, TSLICE)],
                        acc0_out.at[pl.ds(t0, TSLICE)])
        pltpu.sync_copy(den_sh.at[pl.ds(t0, TSLICE)],
                        den0_out.at[pl.ds(t0, TSLICE)])

    @pl.when(c == 1)
    def _():
        pltpu.sync_copy(acc_sh.at[pl.ds(t0, TSLICE)],
                        acc1_out.at[pl.ds(t0, TSLICE)])
        pltpu.sync_copy(den_sh.at[pl.ds(t0, TSLICE)],
                        den1_out.at[pl.ds(t0, TSLICE)])


_edge_pass = pl.kernel(
    _edge_body,
    out_type=(
        jax.ShapeDtypeStruct((NP, D), jnp.float32),
        jax.ShapeDtypeStruct((NP, D), jnp.float32),
        jax.ShapeDtypeStruct((NP,), jnp.float32),
        jax.ShapeDtypeStruct((NP,), jnp.float32),
    ),
    mesh=_MESH,
    compiler_params=pltpu.CompilerParams(needs_layout_passes=False),
    scratch_types=(
        pltpu.VMEM((2, 128), jnp.int32),
        pltpu.VMEM((2, 128), jnp.int32),
        pltpu.VMEM((2, 128), jnp.float32),
        pltpu.VMEM((2, 128), jnp.float32),
        pltpu.VMEM((2, 128), jnp.float32),
        pltpu.VMEM((2, 128, D), jnp.float32),
        pltpu.VMEM_SHARED((NP, D), jnp.float32),
        pltpu.VMEM_SHARED((NP,), jnp.float32),
        pltpu.SemaphoreType.DMA,
        pltpu.SemaphoreType.DMA,
    ),
)

_row_spec = pl.BlockSpec((RB, D), lambda i: (i, 0))
_col_spec = pl.BlockSpec((RB, 1), lambda i: (i, 0))


def _full_spec(r, c):
    return pl.BlockSpec((r, c), lambda i: (0, 0))


def _proj_body(x_ref, w_ref, avs_ref, avd_ref, h_ref, as_ref, ad_ref):
    h = jnp.dot(x_ref[...], w_ref[...], preferred_element_type=jnp.float32)
    h_ref[...] = h
    as_ref[...] = jnp.dot(h, avs_ref[...], preferred_element_type=jnp.float32)
    ad_ref[...] = jnp.dot(h, avd_ref[...], preferred_element_type=jnp.float32)


_proj = pl.pallas_call(
    _proj_body,
    grid=(GRID,),
    in_specs=[_row_spec, _full_spec(D, D), _full_spec(D, 1), _full_spec(D, 1)],
    out_specs=(_row_spec, _col_spec, _col_spec),
    out_shape=(
        jax.ShapeDtypeStruct((NP, D), jnp.float32),
        jax.ShapeDtypeStruct((NP, 1), jnp.float32),
        jax.ShapeDtypeStruct((NP, 1), jnp.float32),
    ),
)


def _combine(h, acc0, acc1, den0, den1, as_c, ad_c):
    e = as_c + ad_c
    e = jnp.where(e >= 0.0, e, 0.2 * e)
    exs = jnp.exp(e)
    num = acc0 + acc1 + exs * h
    dsum = den0 + den1 + exs + 1e-16
    return num / dsum


def _comb_proj_body(h_ref, acc0_ref, acc1_ref, den0_ref, den1_ref,
                    as_ref, ad_ref, b_ref,
                    w_ref, avs_ref, avd_ref, h2_ref, as2_ref, ad2_ref):
    o = _combine(h_ref[...], acc0_ref[...], acc1_ref[...], den0_ref[...],
                 den1_ref[...], as_ref[...], ad_ref[...]) + b_ref[...]
    o = jnp.where(o > 0.0, o, jnp.exp(o) - 1.0)
    h2 = jnp.dot(o, w_ref[...], preferred_element_type=jnp.float32)
    h2_ref[...] = h2
    as2_ref[...] = jnp.dot(h2, avs_ref[...], preferred_element_type=jnp.float32)
    ad2_ref[...] = jnp.dot(h2, avd_ref[...], preferred_element_type=jnp.float32)


_comb_proj = pl.pallas_call(
    _comb_proj_body,
    grid=(GRID,),
    in_specs=[_row_spec, _row_spec, _row_spec, _col_spec, _col_spec,
              _col_spec, _col_spec, _full_spec(1, D),
              _full_spec(D, D), _full_spec(D, 1), _full_spec(D, 1)],
    out_specs=(_row_spec, _col_spec, _col_spec),
    out_shape=(
        jax.ShapeDtypeStruct((NP, D), jnp.float32),
        jax.ShapeDtypeStruct((NP, 1), jnp.float32),
        jax.ShapeDtypeStruct((NP, 1), jnp.float32),
    ),
)


def _final_body(h_ref, acc0_ref, acc1_ref, den0_ref, den1_ref,
                as_ref, ad_ref, b_ref, out_ref):
    out_ref[...] = _combine(
        h_ref[...], acc0_ref[...], acc1_ref[...], den0_ref[...],
        den1_ref[...], as_ref[...], ad_ref[...]) + b_ref[...]


_final = pl.pallas_call(
    _final_body,
    grid=(GRID,),
    in_specs=[_row_spec, _row_spec, _row_spec, _col_spec, _col_spec,
              _col_spec, _col_spec, _full_spec(1, D)],
    out_specs=_row_spec,
    out_shape=jax.ShapeDtypeStruct((NP, D), jnp.float32),
)


def kernel(x, edge_index, W1, a_src1, a_dst1, b1, W2, a_src2, a_dst2, b2):
    xp = jnp.pad(x, ((0, NP - N), (0, 0)))
    src = edge_index[0].astype(jnp.int32)
    dst = edge_index[1].astype(jnp.int32)
    sent = jnp.full((EP - E + 2 * 128,), NP - 1, jnp.int32)
    src2d = jnp.concatenate([src, sent]).reshape(EROWS + 2, 128)
    dst2d = jnp.concatenate([dst, sent]).reshape(EROWS + 2, 128)
    z128 = jnp.zeros((NP, D), jnp.float32)
    z1 = jnp.zeros((NP,), jnp.float32)

    avs1 = a_src1.reshape(D, 1)
    avd1 = a_dst1.reshape(D, 1)
    avs2 = a_src2.reshape(D, 1)
    avd2 = a_dst2.reshape(D, 1)

    h1, as1, ad1 = _proj(xp, W1, avs1, avd1)
    acc10, acc11, den10, den11 = _edge_pass(
        h1, as1.reshape(NP), ad1.reshape(NP), src2d, dst2d, z128, z1)
    h2, as2, ad2 = _comb_proj(
        h1, acc10, acc11, den10.reshape(NP, 1), den11.reshape(NP, 1),
        as1, ad1, b1.reshape(1, D), W2, avs2, avd2)
    acc20, acc21, den20, den21 = _edge_pass(
        h2, as2.reshape(NP), ad2.reshape(NP), src2d, dst2d, z128, z1)
    out = _final(h2, acc20, acc21, den20.reshape(NP, 1), den21.reshape(NP, 1),
                 as2, ad2, b2.reshape(1, D))
    return out[:N]


# speedup vs baseline: 22.2796x; 1.0475x over previous
import jax
import jax.numpy as jnp
from jax import lax
from jax.experimental import pallas as pl
from jax.experimental.pallas import tpu as pltpu
from jax.experimental.pallas import tpu_sc as plsc

N = 10000
D = 128
E = 320000

NP = 10240
EP = 327680
EROWS = EP // 128
A_ROWS = 108
B_ROWS = 52
TSLICE = NP // 16

RB = 1280
GRID = NP // RB

_MESH = plsc.VectorSubcoreMesh(
    core_axis_name="c", subcore_axis_name="s", num_cores=2, num_subcores=16)


def _edge_body(h_hbm, as_hbm, ad_hbm, src_hbm, dst_hbm, z128_hbm, z1_hbm,
               acc0_out, acc1_out, den0_out, den1_out,
               src_v, dst_v, asg_v, adg_v, ex_v, rows_v, acc_sh, den_sh,
               sem0, sem1):
    c = lax.axis_index("c")
    s = lax.axis_index("s")

    t0 = s * TSLICE
    pltpu.sync_copy(z128_hbm.at[pl.ds(t0, TSLICE)], acc_sh.at[pl.ds(t0, TSLICE)])
    pltpu.sync_copy(z1_hbm.at[pl.ds(t0, TSLICE)], den_sh.at[pl.ds(t0, TSLICE)])
    plsc.subcore_barrier()

    row0 = jnp.where(c == 0, s * A_ROWS, 16 * A_ROWS + s * B_ROWS)
    nstep = jnp.where(c == 0, A_ROWS // 2, B_ROWS // 2)
    sems = (sem0, sem1)

    def fire(b, k):
        pltpu.sync_copy(src_hbm.at[row0 + k], src_v.at[b])
        pltpu.sync_copy(dst_hbm.at[row0 + k], dst_v.at[b])
        pltpu.async_copy(h_hbm.at[src_v.at[b]], rows_v.at[b], sems[b])
        pltpu.async_copy(as_hbm.at[src_v.at[b]], asg_v.at[b], sems[b])
        pltpu.async_copy(ad_hbm.at[dst_v.at[b]], adg_v.at[b], sems[b])

    def drain(b):
        pltpu.make_async_copy(h_hbm.at[src_v.at[b]], rows_v.at[b],
                              sems[b]).wait()
        pltpu.make_async_copy(as_hbm.at[src_v.at[b]], asg_v.at[b],
                              sems[b]).wait()
        pltpu.make_async_copy(ad_hbm.at[dst_v.at[b]], adg_v.at[b],
                              sems[b]).wait()

    def process(b):
        for i in range(8):
            sl = pl.ds(i * 16, 16)
            e = asg_v[b, sl] + adg_v[b, sl]
            e = jnp.where(e >= 0.0, e, 0.2 * e)
            ex_v[b, sl] = jnp.exp(e)

        def scale(jg, carry2):
            ex16 = ex_v[b, pl.ds(jg * 16, 16)]
            for l in range(16):
                j = jg * 16 + l
                exj = ex16[l]
                for kk in range(8):
                    sl = pl.ds(kk * 16, 16)
                    rows_v[b, j, sl] = rows_v[b, j, sl] * exj
            return carry2

        lax.fori_loop(0, 8, scale, 0, unroll=2)
        pltpu.sync_copy(rows_v.at[b], acc_sh.at[dst_v.at[b]], add=True)
        pltpu.sync_copy(ex_v.at[b], den_sh.at[dst_v.at[b]], add=True)

    fire(0, 0)
    fire(1, 1)

    def step(k2, carry):
        k = k2 * 2
        for b in range(2):
            drain(b)
            process(b)
            fire(b, k + b + 2)
        return carry

    lax.fori_loop(0, nstep, step, 0)
    drain(0)
    drain(1)
    plsc.subcore_barrier()

    @pl.when(c == 0)
    def _():
        pltpu.sync_copy(acc_sh.at[pl.ds(t0, TSLICE)],
                        acc0_out.at[pl.ds(t0, TSLICE)])
        pltpu.sync_copy(den_sh.at[pl.ds(t0, TSLICE)],
                        den0_out.at[pl.ds(t0, TSLICE)])

    @pl.when(c == 1)
    def _():
        pltpu.sync_copy(acc_sh.at[pl.ds(t0, TSLICE)],
                        acc1_out.at[pl.ds(t0, TSLICE)])
        pltpu.sync_copy(den_sh.at[pl.ds(t0, TSLICE)],
                        den1_out.at[pl.ds(t0, TSLICE)])


_edge_pass = pl.kernel(
    _edge_body,
    out_type=(
        jax.ShapeDtypeStruct((NP, D), jnp.float32),
        jax.ShapeDtypeStruct((NP, D), jnp.float32),
        jax.ShapeDtypeStruct((NP,), jnp.float32),
        jax.ShapeDtypeStruct((NP,), jnp.float32),
    ),
    mesh=_MESH,
    compiler_params=pltpu.CompilerParams(needs_layout_passes=False),
    scratch_types=(
        pltpu.VMEM((2, 128), jnp.int32),
        pltpu.VMEM((2, 128), jnp.int32),
        pltpu.VMEM((2, 128), jnp.float32),
        pltpu.VMEM((2, 128), jnp.float32),
        pltpu.VMEM((2, 128), jnp.float32),
        pltpu.VMEM((2, 128, D), jnp.float32),
        pltpu.VMEM_SHARED((NP, D), jnp.float32),
        pltpu.VMEM_SHARED((NP,), jnp.float32),
        pltpu.SemaphoreType.DMA,
        pltpu.SemaphoreType.DMA,
    ),
)

_row_spec = pl.BlockSpec((RB, D), lambda i: (i, 0))
_col_spec = pl.BlockSpec((RB, 1), lambda i: (i, 0))


def _full_spec(r, c):
    return pl.BlockSpec((r, c), lambda i: (0, 0))


def _proj_body(x_ref, w_ref, avs_ref, avd_ref, h_ref, as_ref, ad_ref):
    h = jnp.dot(x_ref[...], w_ref[...], preferred_element_type=jnp.float32)
    h_ref[...] = h
    as_ref[...] = jnp.dot(h, avs_ref[...], preferred_element_type=jnp.float32)
    ad_ref[...] = jnp.dot(h, avd_ref[...], preferred_element_type=jnp.float32)


_proj = pl.pallas_call(
    _proj_body,
    grid=(GRID,),
    in_specs=[_row_spec, _full_spec(D, D), _full_spec(D, 1), _full_spec(D, 1)],
    out_specs=(_row_spec, _col_spec, _col_spec),
    out_shape=(
        jax.ShapeDtypeStruct((NP, D), jnp.float32),
        jax.ShapeDtypeStruct((NP, 1), jnp.float32),
        jax.ShapeDtypeStruct((NP, 1), jnp.float32),
    ),
)


def _combine(h, acc0, acc1, den0, den1, as_c, ad_c):
    e = as_c + ad_c
    e = jnp.where(e >= 0.0, e, 0.2 * e)
    exs = jnp.exp(e)
    num = acc0 + acc1 + exs * h
    dsum = den0 + den1 + exs + 1e-16
    return num / dsum


def _comb_proj_body(h_ref, acc0_ref, acc1_ref, den0_ref, den1_ref,
                    as_ref, ad_ref, b_ref,
                    w_ref, avs_ref, avd_ref, h2_ref, as2_ref, ad2_ref):
    o = _combine(h_ref[...], acc0_ref[...], acc1_ref[...], den0_ref[...],
                 den1_ref[...], as_ref[...], ad_ref[...]) + b_ref[...]
    o = jnp.where(o > 0.0, o, jnp.exp(o) - 1.0)
    h2 = jnp.dot(o, w_ref[...], preferred_element_type=jnp.float32)
    h2_ref[...] = h2
    as2_ref[...] = jnp.dot(h2, avs_ref[...], preferred_element_type=jnp.float32)
    ad2_ref[...] = jnp.dot(h2, avd_ref[...], preferred_element_type=jnp.float32)


_comb_proj = pl.pallas_call(
    _comb_proj_body,
    grid=(GRID,),
    in_specs=[_row_spec, _row_spec, _row_spec, _col_spec, _col_spec,
              _col_spec, _col_spec, _full_spec(1, D),
              _full_spec(D, D), _full_spec(D, 1), _full_spec(D, 1)],
    out_specs=(_row_spec, _col_spec, _col_spec),
    out_shape=(
        jax.ShapeDtypeStruct((NP, D), jnp.float32),
        jax.ShapeDtypeStruct((NP, 1), jnp.float32),
        jax.ShapeDtypeStruct((NP, 1), jnp.float32),
    ),
)


def _final_body(h_ref, acc0_ref, acc1_ref, den0_ref, den1_ref,
                as_ref, ad_ref, b_ref, out_ref):
    out_ref[...] = _combine(
        h_ref[...], acc0_ref[...], acc1_ref[...], den0_ref[...],
        den1_ref[...], as_ref[...], ad_ref[...]) + b_ref[...]


_final = pl.pallas_call(
    _final_body,
    grid=(GRID,),
    in_specs=[_row_spec, _row_spec, _row_spec, _col_spec, _col_spec,
              _col_spec, _col_spec, _full_spec(1, D)],
    out_specs=_row_spec,
    out_shape=jax.ShapeDtypeStruct((NP, D), jnp.float32),
)


def kernel(x, edge_index, W1, a_src1, a_dst1, b1, W2, a_src2, a_dst2, b2):
    xp = jnp.pad(x, ((0, NP - N), (0, 0)))
    src = edge_index[0].astype(jnp.int32)
    dst = edge_index[1].astype(jnp.int32)
    sent = jnp.full((EP - E + 2 * 128,), NP - 1, jnp.int32)
    src2d = jnp.concatenate([src, sent]).reshape(EROWS + 2, 128)
    dst2d = jnp.concatenate([dst, sent]).reshape(EROWS + 2, 128)
    z128 = jnp.zeros((NP, D), jnp.float32)
    z1 = jnp.zeros((NP,), jnp.float32)

    avs1 = a_src1.reshape(D, 1)
    avd1 = a_dst1.reshape(D, 1)
    avs2 = a_src2.reshape(D, 1)
    avd2 = a_dst2.reshape(D, 1)

    h1, as1, ad1 = _proj(xp, W1, avs1, avd1)
    acc10, acc11, den10, den11 = _edge_pass(
        h1, as1.reshape(NP), ad1.reshape(NP), src2d, dst2d, z128, z1)
    h2, as2, ad2 = _comb_proj(
        h1, acc10, acc11, den10.reshape(NP, 1), den11.reshape(NP, 1),
        as1, ad1, b1.reshape(1, D), W2, avs2, avd2)
    acc20, acc21, den20, den21 = _edge_pass(
        h2, as2.reshape(NP), ad2.reshape(NP), src2d, dst2d, z128, z1)
    out = _final(h2, acc20, acc21, den20.reshape(NP, 1), den21.reshape(NP, 1),
                 as2, ad2, b2.reshape(1, D))
    return out[:N]


# speedup vs baseline: 22.9252x; 1.0290x over previous
import jax
import jax.numpy as jnp
from jax import lax
from jax.experimental import pallas as pl
from jax.experimental.pallas import tpu as pltpu
from jax.experimental.pallas import tpu_sc as plsc

N = 10000
D = 128
E = 320000

NP = 10240
EP = 327680
EROWS = EP // 128
A_ROWS = 120
B_ROWS = 40
TSLICE = NP // 16

RB = 1280
GRID = NP // RB

_MESH = plsc.VectorSubcoreMesh(
    core_axis_name="c", subcore_axis_name="s", num_cores=2, num_subcores=16)


def _edge_body(h_hbm, as_hbm, ad_hbm, src_hbm, dst_hbm,
               acc0_out, acc1_out, den0_out, den1_out,
               src_v, dst_v, asg_v, adg_v, ex_v, rows_v, acc_sh, den_sh,
               sem0, sem1):
    c = lax.axis_index("c")
    s = lax.axis_index("s")

    zero16 = jnp.zeros((16,), jnp.float32)
    for b in range(2):
        for i in range(8):
            ex_v[b, pl.ds(i * 16, 16)] = zero16

    def zrow(j, carry):
        for kk in range(8):
            rows_v[0, j, pl.ds(kk * 16, 16)] = zero16
        return carry

    lax.fori_loop(0, 128, zrow, 0)
    t0 = s * TSLICE
    for i in range(TSLICE // 128):
        pltpu.sync_copy(rows_v.at[0], acc_sh.at[pl.ds(t0 + i * 128, 128)])
        pltpu.sync_copy(ex_v.at[0], den_sh.at[pl.ds(t0 + i * 128, 128)])
    plsc.subcore_barrier()

    row0 = jnp.where(c == 0, s * A_ROWS, 16 * A_ROWS + s * B_ROWS)
    nstep = jnp.where(c == 0, A_ROWS // 2, B_ROWS // 2)
    sems = (sem0, sem1)

    def fire(b, k):
        pltpu.sync_copy(src_hbm.at[row0 + k], src_v.at[b])
        pltpu.sync_copy(dst_hbm.at[row0 + k], dst_v.at[b])
        pltpu.async_copy(h_hbm.at[src_v.at[b]], rows_v.at[b], sems[b])
        pltpu.async_copy(as_hbm.at[src_v.at[b]], asg_v.at[b], sems[b])
        pltpu.async_copy(ad_hbm.at[dst_v.at[b]], adg_v.at[b], sems[b])

    def drain(b):
        pltpu.make_async_copy(h_hbm.at[src_v.at[b]], rows_v.at[b],
                              sems[b]).wait()
        pltpu.make_async_copy(as_hbm.at[src_v.at[b]], asg_v.at[b],
                              sems[b]).wait()
        pltpu.make_async_copy(ad_hbm.at[dst_v.at[b]], adg_v.at[b],
                              sems[b]).wait()

    def process(b):
        for i in range(8):
            sl = pl.ds(i * 16, 16)
            e = asg_v[b, sl] + adg_v[b, sl]
            e = jnp.where(e >= 0.0, e, 0.2 * e)
            ex_v[b, sl] = jnp.exp(e)

        def scale(jg, carry2):
            ex16 = ex_v[b, pl.ds(jg * 16, 16)]
            for l in range(16):
                j = jg * 16 + l
                exj = ex16[l]
                for kk in range(8):
                    sl = pl.ds(kk * 16, 16)
                    rows_v[b, j, sl] = rows_v[b, j, sl] * exj
            return carry2

        lax.fori_loop(0, 8, scale, 0, unroll=2)
        pltpu.sync_copy(rows_v.at[b], acc_sh.at[dst_v.at[b]], add=True)
        pltpu.sync_copy(ex_v.at[b], den_sh.at[dst_v.at[b]], add=True)

    fire(0, 0)
    fire(1, 1)

    def step(k2, carry):
        k = k2 * 2
        for b in range(2):
            drain(b)
            process(b)
            fire(b, k + b + 2)
        return carry

    lax.fori_loop(0, nstep, step, 0)
    drain(0)
    drain(1)
    plsc.subcore_barrier()

    @pl.when(c == 0)
    def _():
        pltpu.sync_copy(acc_sh.at[pl.ds(t0, TSLICE)],
                        acc0_out.at[pl.ds(t0, TSLICE)])
        pltpu.sync_copy(den_sh.at[pl.ds(t0, TSLICE)],
                        den0_out.at[pl.ds(t0, TSLICE)])

    @pl.when(c == 1)
    def _():
        pltpu.sync_copy(acc_sh.at[pl.ds(t0, TSLICE)],
                        acc1_out.at[pl.ds(t0, TSLICE)])
        pltpu.sync_copy(den_sh.at[pl.ds(t0, TSLICE)],
                        den1_out.at[pl.ds(t0, TSLICE)])


_edge_pass = pl.kernel(
    _edge_body,
    out_type=(
        jax.ShapeDtypeStruct((NP, D), jnp.float32),
        jax.ShapeDtypeStruct((NP, D), jnp.float32),
        jax.ShapeDtypeStruct((NP,), jnp.float32),
        jax.ShapeDtypeStruct((NP,), jnp.float32),
    ),
    mesh=_MESH,
    compiler_params=pltpu.CompilerParams(needs_layout_passes=False),
    scratch_types=(
        pltpu.VMEM((2, 128), jnp.int32),
        pltpu.VMEM((2, 128), jnp.int32),
        pltpu.VMEM((2, 128), jnp.float32),
        pltpu.VMEM((2, 128), jnp.float32),
        pltpu.VMEM((2, 128), jnp.float32),
        pltpu.VMEM((2, 128, D), jnp.float32),
        pltpu.VMEM_SHARED((NP, D), jnp.float32),
        pltpu.VMEM_SHARED((NP,), jnp.float32),
        pltpu.SemaphoreType.DMA,
        pltpu.SemaphoreType.DMA,
    ),
)

_row_spec = pl.BlockSpec((RB, D), lambda i: (i, 0))
_col_spec = pl.BlockSpec((RB, 1), lambda i: (i, 0))


def _full_spec(r, c):
    return pl.BlockSpec((r, c), lambda i: (0, 0))


def _proj_body(x_ref, w_ref, avs_ref, avd_ref, h_ref, as_ref, ad_ref):
    h = jnp.dot(x_ref[...], w_ref[...], preferred_element_type=jnp.float32)
    h_ref[...] = h
    as_ref[...] = jnp.dot(h, avs_ref[...], preferred_element_type=jnp.float32)
    ad_ref[...] = jnp.dot(h, avd_ref[...], preferred_element_type=jnp.float32)


_proj = pl.pallas_call(
    _proj_body,
    grid=(GRID,),
    in_specs=[_row_spec, _full_spec(D, D), _full_spec(D, 1), _full_spec(D, 1)],
    out_specs=(_row_spec, _col_spec, _col_spec),
    out_shape=(
        jax.ShapeDtypeStruct((NP, D), jnp.float32),
        jax.ShapeDtypeStruct((NP, 1), jnp.float32),
        jax.ShapeDtypeStruct((NP, 1), jnp.float32),
    ),
)


def _combine(h, acc0, acc1, den0, den1, as_c, ad_c):
    e = as_c + ad_c
    e = jnp.where(e >= 0.0, e, 0.2 * e)
    exs = jnp.exp(e)
    num = acc0 + acc1 + exs * h
    dsum = den0 + den1 + exs + 1e-16
    return num / dsum


def _comb_proj_body(h_ref, acc0_ref, acc1_ref, den0_ref, den1_ref,
                    as_ref, ad_ref, b_ref,
                    w_ref, avs_ref, avd_ref, h2_ref, as2_ref, ad2_ref):
    o = _combine(h_ref[...], acc0_ref[...], acc1_ref[...], den0_ref[...],
                 den1_ref[...], as_ref[...], ad_ref[...]) + b_ref[...]
    o = jnp.where(o > 0.0, o, jnp.exp(o) - 1.0)
    h2 = jnp.dot(o, w_ref[...], preferred_element_type=jnp.float32)
    h2_ref[...] = h2
    as2_ref[...] = jnp.dot(h2, avs_ref[...], preferred_element_type=jnp.float32)
    ad2_ref[...] = jnp.dot(h2, avd_ref[...], preferred_element_type=jnp.float32)


_comb_proj = pl.pallas_call(
    _comb_proj_body,
    grid=(GRID,),
    in_specs=[_row_spec, _row_spec, _row_spec, _col_spec, _col_spec,
              _col_spec, _col_spec, _full_spec(1, D),
              _full_spec(D, D), _full_spec(D, 1), _full_spec(D, 1)],
    out_specs=(_row_spec, _col_spec, _col_spec),
    out_shape=(
        jax.ShapeDtypeStruct((NP, D), jnp.float32),
        jax.ShapeDtypeStruct((NP, 1), jnp.float32),
        jax.ShapeDtypeStruct((NP, 1), jnp.float32),
    ),
)


def _final_body(h_ref, acc0_ref, acc1_ref, den0_ref, den1_ref,
                as_ref, ad_ref, b_ref, out_ref):
    out_ref[...] = _combine(
        h_ref[...], acc0_ref[...], acc1_ref[...], den0_ref[...],
        den1_ref[...], as_ref[...], ad_ref[...]) + b_ref[...]


_final = pl.pallas_call(
    _final_body,
    grid=(GRID,),
    in_specs=[_row_spec, _row_spec, _row_spec, _col_spec, _col_spec,
              _col_spec, _col_spec, _full_spec(1, D)],
    out_specs=_row_spec,
    out_shape=jax.ShapeDtypeStruct((NP, D), jnp.float32),
)


def kernel(x, edge_index, W1, a_src1, a_dst1, b1, W2, a_src2, a_dst2, b2):
    xp = jnp.pad(x, ((0, NP - N), (0, 0)))
    src = edge_index[0].astype(jnp.int32)
    dst = edge_index[1].astype(jnp.int32)
    sent = jnp.full((EP - E + 2 * 128,), NP - 1, jnp.int32)
    src2d = jnp.concatenate([src, sent]).reshape(EROWS + 2, 128)
    dst2d = jnp.concatenate([dst, sent]).reshape(EROWS + 2, 128)

    avs1 = a_src1.reshape(D, 1)
    avd1 = a_dst1.reshape(D, 1)
    avs2 = a_src2.reshape(D, 1)
    avd2 = a_dst2.reshape(D, 1)

    h1, as1, ad1 = _proj(xp, W1, avs1, avd1)
    acc10, acc11, den10, den11 = _edge_pass(
        h1, as1.reshape(NP), ad1.reshape(NP), src2d, dst2d)
    h2, as2, ad2 = _comb_proj(
        h1, acc10, acc11, den10.reshape(NP, 1), den11.reshape(NP, 1),
        as1, ad1, b1.reshape(1, D), W2, avs2, avd2)
    acc20, acc21, den20, den21 = _edge_pass(
        h2, as2.reshape(NP), ad2.reshape(NP), src2d, dst2d)
    out = _final(h2, acc20, acc21, den20.reshape(NP, 1), den21.reshape(NP, 1),
                 as2, ad2, b2.reshape(1, D))
    return out[:N]


# speedup vs baseline: 39.8088x; 1.7365x over previous
import jax
import jax.numpy as jnp
from jax import lax
from jax.experimental import pallas as pl
from jax.experimental.pallas import tpu as pltpu
from jax.experimental.pallas import tpu_sc as plsc

N = 10000
D = 128
E = 320000

NP = 10240
EP = 327680
EROWS = EP // 128
A_ROWS = 80
B_ROWS = 80
TSLICE = NP // 16

RB = 1280
GRID = NP // RB

_MESH = plsc.VectorSubcoreMesh(
    core_axis_name="c", subcore_axis_name="s", num_cores=2, num_subcores=16)


def _edge_body(h_hbm, as_hbm, ad_hbm, src_hbm, dst_hbm,
               acc0_out, acc1_out, den0_out, den1_out,
               src_v, dst_v, asg_v, adg_v, ex_v, rows_v, acc_sh, den_sh,
               sem0, sem1):
    c = lax.axis_index("c")
    s = lax.axis_index("s")

    zero16 = jnp.zeros((16,), jnp.float32)
    for b in range(2):
        for i in range(8):
            ex_v[b, pl.ds(i * 16, 16)] = zero16

    def zrow(j, carry):
        for kk in range(8):
            rows_v[0, j, pl.ds(kk * 16, 16)] = zero16
        return carry

    lax.fori_loop(0, 128, zrow, 0)
    t0 = s * TSLICE
    for i in range(TSLICE // 128):
        pltpu.sync_copy(rows_v.at[0], acc_sh.at[pl.ds(t0 + i * 128, 128)])
        pltpu.sync_copy(ex_v.at[0], den_sh.at[pl.ds(t0 + i * 128, 128)])
    plsc.subcore_barrier()

    row0 = jnp.where(c == 0, s * A_ROWS, 16 * A_ROWS + s * B_ROWS)
    nstep = jnp.where(c == 0, A_ROWS // 2, B_ROWS // 2)
    sems = (sem0, sem1)

    def fire(b, k):
        pltpu.sync_copy(src_hbm.at[row0 + k], src_v.at[b])
        pltpu.sync_copy(dst_hbm.at[row0 + k], dst_v.at[b])
        pltpu.async_copy(h_hbm.at[src_v.at[b]], rows_v.at[b], sems[b])
        pltpu.async_copy(as_hbm.at[src_v.at[b]], asg_v.at[b], sems[b])
        pltpu.async_copy(ad_hbm.at[dst_v.at[b]], adg_v.at[b], sems[b])

    def drain(b):
        pltpu.make_async_copy(h_hbm.at[src_v.at[b]], rows_v.at[b],
                              sems[b]).wait()
        pltpu.make_async_copy(as_hbm.at[src_v.at[b]], asg_v.at[b],
                              sems[b]).wait()
        pltpu.make_async_copy(ad_hbm.at[dst_v.at[b]], adg_v.at[b],
                              sems[b]).wait()

    def process(b):
        for i in range(8):
            sl = pl.ds(i * 16, 16)
            e = asg_v[b, sl] + adg_v[b, sl]
            e = jnp.where(e >= 0.0, e, 0.2 * e)
            ex_v[b, sl] = jnp.exp(e)

        def scale(jg, carry2):
            ex16 = ex_v[b, pl.ds(jg * 16, 16)]
            for l in range(16):
                j = jg * 16 + l
                exj = ex16[l]
                for kk in range(8):
                    sl = pl.ds(kk * 16, 16)
                    rows_v[b, j, sl] = rows_v[b, j, sl] * exj
            return carry2

        lax.fori_loop(0, 8, scale, 0, unroll=2)
        pltpu.sync_copy(rows_v.at[b], acc_sh.at[dst_v.at[b]], add=True)
        pltpu.sync_copy(ex_v.at[b], den_sh.at[dst_v.at[b]], add=True)

    fire(0, 0)
    fire(1, 1)

    def step(k2, carry):
        k = k2 * 2
        for b in range(2):
            drain(b)
            process(b)
            fire(b, k + b + 2)
        return carry

    lax.fori_loop(0, nstep, step, 0)
    drain(0)
    drain(1)
    plsc.subcore_barrier()

    @pl.when(c == 0)
    def _():
        pltpu.sync_copy(acc_sh.at[pl.ds(t0, TSLICE)],
                        acc0_out.at[pl.ds(t0, TSLICE)])
        pltpu.sync_copy(den_sh.at[pl.ds(t0, TSLICE)],
                        den0_out.at[pl.ds(t0, TSLICE)])

    @pl.when(c == 1)
    def _():
        pltpu.sync_copy(acc_sh.at[pl.ds(t0, TSLICE)],
                        acc1_out.at[pl.ds(t0, TSLICE)])
        pltpu.sync_copy(den_sh.at[pl.ds(t0, TSLICE)],
                        den1_out.at[pl.ds(t0, TSLICE)])


_edge_pass = pl.kernel(
    _edge_body,
    out_type=(
        jax.ShapeDtypeStruct((NP, D), jnp.float32),
        jax.ShapeDtypeStruct((NP, D), jnp.float32),
        jax.ShapeDtypeStruct((NP,), jnp.float32),
        jax.ShapeDtypeStruct((NP,), jnp.float32),
    ),
    mesh=_MESH,
    compiler_params=pltpu.CompilerParams(needs_layout_passes=False),
    scratch_types=(
        pltpu.VMEM((2, 128), jnp.int32),
        pltpu.VMEM((2, 128), jnp.int32),
        pltpu.VMEM((2, 128), jnp.float32),
        pltpu.VMEM((2, 128), jnp.float32),
        pltpu.VMEM((2, 128), jnp.float32),
        pltpu.VMEM((2, 128, D), jnp.float32),
        pltpu.VMEM_SHARED((NP, D), jnp.float32),
        pltpu.VMEM_SHARED((NP,), jnp.float32),
        pltpu.SemaphoreType.DMA,
        pltpu.SemaphoreType.DMA,
    ),
)

_row_spec = pl.BlockSpec((RB, D), lambda i: (i, 0))
_col_spec = pl.BlockSpec((RB, 1), lambda i: (i, 0))


def _full_spec(r, c):
    return pl.BlockSpec((r, c), lambda i: (0, 0))


def _proj_body(x_ref, w_ref, avs_ref, avd_ref, h_ref, as_ref, ad_ref):
    h = jnp.dot(x_ref[...], w_ref[...], preferred_element_type=jnp.float32)
    h_ref[...] = h
    as_ref[...] = jnp.dot(h, avs_ref[...], preferred_element_type=jnp.float32)
    ad_ref[...] = jnp.dot(h, avd_ref[...], preferred_element_type=jnp.float32)


_proj = pl.pallas_call(
    _proj_body,
    grid=(GRID,),
    in_specs=[_row_spec, _full_spec(D, D), _full_spec(D, 1), _full_spec(D, 1)],
    out_specs=(_row_spec, _col_spec, _col_spec),
    out_shape=(
        jax.ShapeDtypeStruct((NP, D), jnp.float32),
        jax.ShapeDtypeStruct((NP, 1), jnp.float32),
        jax.ShapeDtypeStruct((NP, 1), jnp.float32),
    ),
)


def _combine(h, acc0, acc1, den0, den1, as_c, ad_c):
    e = as_c + ad_c
    e = jnp.where(e >= 0.0, e, 0.2 * e)
    exs = jnp.exp(e)
    num = acc0 + acc1 + exs * h
    dsum = den0 + den1 + exs + 1e-16
    return num / dsum


def _comb_proj_body(h_ref, acc0_ref, acc1_ref, den0_ref, den1_ref,
                    as_ref, ad_ref, b_ref,
                    w_ref, avs_ref, avd_ref, h2_ref, as2_ref, ad2_ref):
    o = _combine(h_ref[...], acc0_ref[...], acc1_ref[...], den0_ref[...],
                 den1_ref[...], as_ref[...], ad_ref[...]) + b_ref[...]
    o = jnp.where(o > 0.0, o, jnp.exp(o) - 1.0)
    h2 = jnp.dot(o, w_ref[...], preferred_element_type=jnp.float32)
    h2_ref[...] = h2
    as2_ref[...] = jnp.dot(h2, avs_ref[...], preferred_element_type=jnp.float32)
    ad2_ref[...] = jnp.dot(h2, avd_ref[...], preferred_element_type=jnp.float32)


_comb_proj = pl.pallas_call(
    _comb_proj_body,
    grid=(GRID,),
    in_specs=[_row_spec, _row_spec, _row_spec, _col_spec, _col_spec,
              _col_spec, _col_spec, _full_spec(1, D),
              _full_spec(D, D), _full_spec(D, 1), _full_spec(D, 1)],
    out_specs=(_row_spec, _col_spec, _col_spec),
    out_shape=(
        jax.ShapeDtypeStruct((NP, D), jnp.float32),
        jax.ShapeDtypeStruct((NP, 1), jnp.float32),
        jax.ShapeDtypeStruct((NP, 1), jnp.float32),
    ),
)


def _final_body(h_ref, acc0_ref, acc1_ref, den0_ref, den1_ref,
                as_ref, ad_ref, b_ref, out_ref):
    out_ref[...] = _combine(
        h_ref[...], acc0_ref[...], acc1_ref[...], den0_ref[...],
        den1_ref[...], as_ref[...], ad_ref[...]) + b_ref[...]


_final = pl.pallas_call(
    _final_body,
    grid=(GRID,),
    in_specs=[_row_spec, _row_spec, _row_spec, _col_spec, _col_spec,
              _col_spec, _col_spec, _full_spec(1, D)],
    out_specs=_row_spec,
    out_shape=jax.ShapeDtypeStruct((NP, D), jnp.float32),
)


def kernel(x, edge_index, W1, a_src1, a_dst1, b1, W2, a_src2, a_dst2, b2):
    xp = jnp.pad(x, ((0, NP - N), (0, 0)))
    src = edge_index[0].astype(jnp.int32)
    dst = edge_index[1].astype(jnp.int32)
    sent = (NP - 128) + (jnp.arange(EP - E + 2 * 128, dtype=jnp.int32) % 128)
    src2d = jnp.concatenate([src, sent]).reshape(EROWS + 2, 128)
    dst2d = jnp.concatenate([dst, sent]).reshape(EROWS + 2, 128)

    avs1 = a_src1.reshape(D, 1)
    avd1 = a_dst1.reshape(D, 1)
    avs2 = a_src2.reshape(D, 1)
    avd2 = a_dst2.reshape(D, 1)

    h1, as1, ad1 = _proj(xp, W1, avs1, avd1)
    acc10, acc11, den10, den11 = _edge_pass(
        h1, as1.reshape(NP), ad1.reshape(NP), src2d, dst2d)
    h2, as2, ad2 = _comb_proj(
        h1, acc10, acc11, den10.reshape(NP, 1), den11.reshape(NP, 1),
        as1, ad1, b1.reshape(1, D), W2, avs2, avd2)
    acc20, acc21, den20, den21 = _edge_pass(
        h2, as2.reshape(NP), ad2.reshape(NP), src2d, dst2d)
    out = _final(h2, acc20, acc21, den20.reshape(NP, 1), den21.reshape(NP, 1),
                 as2, ad2, b2.reshape(1, D))
    return out[:N]
